# baseline (speedup 1.0000x reference)
import functools

import jax
import jax.numpy as jnp
from jax import lax
from jax.experimental import pallas as pl
from jax.experimental.pallas import tpu as pltpu
from jax.experimental.pallas import tpu_sc as plsc

N = 10000
E = 160000
DIN = 256
D = 512
G = 64

NC = 2
NS = 16
CHUNK = 128
ECHUNK = 128
EPT = 10240
EPAD = EPT * NS
NCHUNKS = EPT // ECHUNK
NSEG = 10
SEG = NCHUNKS // NSEG
RPT = 624
REM = N - NS * RPT
SLAB_ROWS = N + 16

BR = 1000
RING = 2


def _sc_agg_body(nq, hflat_hbm, gidx_hbm, didx_hbm, z_hbm,
                 gidx_v, didx_v, gbuf, slab, *sems):
    gsem = sems[:RING]
    ssem = sems[RING:]
    c = lax.axis_index("c")
    s = lax.axis_index("s")
    r0 = s * RPT
    passes = nq // NC
    for p in range(passes):
        q = p * NC + c
        hrow0 = pl.multiple_of(q * N + r0, 8)
        pltpu.sync_copy(hflat_hbm.at[pl.ds(hrow0, RPT)],
                        slab.at[pl.ds(r0, RPT)])

        @pl.when(s == NS - 1)
        def _():
            pltpu.sync_copy(
                hflat_hbm.at[pl.ds(pl.multiple_of(q * N + NS * RPT, 8), REM)],
                slab.at[pl.ds(NS * RPT, REM)])
        plsc.subcore_barrier()

        for seg in range(NSEG):
            segbase = s * NCHUNKS + seg * SEG
            pltpu.sync_copy(gidx_hbm.at[q, pl.ds(segbase, SEG)], gidx_v)
            pltpu.sync_copy(didx_hbm.at[pl.ds(segbase, SEG)], didx_v)
            for r in range(RING):
                pltpu.async_copy(hflat_hbm.at[gidx_v.at[r]], gbuf.at[r],
                                 gsem[r])

            def body(m, carry):
                for r in range(RING):
                    j = m * RING + r
                    pltpu.make_async_copy(hflat_hbm.at[pl.ds(0, ECHUNK)],
                                          gbuf.at[r], gsem[r]).wait()
                    pltpu.async_copy(gbuf.at[r], slab.at[didx_v.at[j]],
                                     ssem[r], add=True)
                for r in range(RING):
                    j = m * RING + r
                    pltpu.make_async_copy(gbuf.at[r], slab.at[pl.ds(0, ECHUNK)],
                                          ssem[r]).wait()
                    jn = jnp.minimum(j + RING, SEG - 1)
                    pltpu.async_copy(hflat_hbm.at[gidx_v.at[jn]], gbuf.at[r],
                                     gsem[r])
                return carry

            lax.fori_loop(0, SEG // RING, body, 0)
            for r in range(RING):
                pltpu.make_async_copy(hflat_hbm.at[pl.ds(0, ECHUNK)],
                                      gbuf.at[r], gsem[r]).wait()
        plsc.subcore_barrier()
        pltpu.sync_copy(slab.at[pl.ds(r0, RPT)], z_hbm.at[q, pl.ds(r0, RPT)])

        @pl.when(s == NS - 1)
        def _():
            pltpu.sync_copy(slab.at[pl.ds(NS * RPT, REM)],
                            z_hbm.at[q, pl.ds(NS * RPT, REM)])


@functools.cache
def _make_sc_agg(nq):
    mesh = plsc.VectorSubcoreMesh(core_axis_name="c", subcore_axis_name="s",
                                  num_cores=NC, num_subcores=NS)
    return functools.partial(
        pl.kernel,
        out_type=jax.ShapeDtypeStruct((nq, N, CHUNK), jnp.float32),
        mesh=mesh,
        scratch_types=[
            pltpu.VMEM((SEG, ECHUNK), jnp.int32),
            pltpu.VMEM((SEG, ECHUNK), jnp.int32),
            pltpu.VMEM((RING, ECHUNK, CHUNK), jnp.float32),
            pltpu.VMEM_SHARED((SLAB_ROWS, CHUNK), jnp.float32),
        ] + [pltpu.SemaphoreType.DMA] * (2 * RING) + [
        ],
    )(functools.partial(_sc_agg_body, nq))


def _mm_body(z_ref, w_ref, b_ref, out_ref, *, nq):
    s = jnp.dot(z_ref[0], w_ref[0], preferred_element_type=jnp.float32)
    for q in range(1, nq):
        s += jnp.dot(z_ref[q], w_ref[q], preferred_element_type=jnp.float32)
    h = jnp.maximum(s + b_ref[...], 0.0)
    for qo in range(D // 128):
        out_ref[qo] = h[:, qo * 128:(qo + 1) * 128]


def _layer_mm(z, wv, b2d, nq):
    return pl.pallas_call(
        functools.partial(_mm_body, nq=nq),
        grid=(N // BR,),
        in_specs=[
            pl.BlockSpec((nq, BR, 128), lambda i: (0, i, 0)),
            pl.BlockSpec((nq, 128, D), lambda i: (0, 0, 0)),
            pl.BlockSpec((1, D), lambda i: (0, 0)),
        ],
        out_specs=pl.BlockSpec((D // 128, BR, 128), lambda i: (0, i, 0)),
        out_shape=jax.ShapeDtypeStruct((D // 128, N, 128), jnp.float32),
    )(z, wv, b2d)


def _mm3_body(z_ref, w_ref, b_ref, bi_ref, out_ref, *, nq):
    s = jnp.dot(z_ref[0], w_ref[0], preferred_element_type=jnp.float32)
    for q in range(1, nq):
        s += jnp.dot(z_ref[q], w_ref[q], preferred_element_type=jnp.float32)
    h = jnp.maximum(s + b_ref[...], 0.0)
    hext = jnp.concatenate([h, jnp.ones((BR, 128), jnp.float32)], axis=1)
    bidx = bi_ref[0, 0, :]
    oh = (lax.broadcasted_iota(jnp.int32, (G, BR), 0) == bidx[None, :])
    contrib = jnp.dot(oh.astype(jnp.float32), hext,
                      preferred_element_type=jnp.float32)
    i = pl.program_id(0)

    @pl.when(i == 0)
    def _():
        out_ref[...] = contrib

    @pl.when(i != 0)
    def _():
        out_ref[...] = out_ref[...] + contrib


def _layer3_pool(z, wv, b2d, bidx3, nq):
    return pl.pallas_call(
        functools.partial(_mm3_body, nq=nq),
        grid=(N // BR,),
        in_specs=[
            pl.BlockSpec((nq, BR, 128), lambda i: (0, i, 0)),
            pl.BlockSpec((nq, 128, D), lambda i: (0, 0, 0)),
            pl.BlockSpec((1, D), lambda i: (0, 0)),
            pl.BlockSpec((1, 1, BR), lambda i: (i, 0, 0)),
        ],
        out_specs=pl.BlockSpec((G, D + 128), lambda i: (0, 0)),
        out_shape=jax.ShapeDtypeStruct((G, D + 128), jnp.float32),
    )(z, wv, b2d, bidx3)


def _post_body(s_ref, p1_ref, pb1_ref, p2_ref, pb2_ref, out_ref):
    sums = s_ref[:, :D]
    counts = s_ref[:, D:]
    recip = 1.0 / jnp.maximum(counts, 1.0)
    recip_full = jnp.concatenate([recip] * (D // 128), axis=1)
    pooled = sums * recip_full
    t = jnp.maximum(
        jnp.dot(pooled, p1_ref[...], preferred_element_type=jnp.float32)
        + pb1_ref[...], 0.0)
    out_ref[...] = jnp.maximum(
        jnp.dot(t, p2_ref[...], preferred_element_type=jnp.float32)
        + pb2_ref[...], 0.0)


def _post(sums, p1, pb1, p2, pb2):
    return pl.pallas_call(
        _post_body,
        out_shape=jax.ShapeDtypeStruct((G, D), jnp.float32),
    )(sums, p1, pb1.reshape(1, D), p2, pb2.reshape(1, D))


def kernel(x, edge_index, batch_index, W1, b1, W2, b2, W3, b3, P1, pb1, P2, pb2):
    src = edge_index[0]
    dst = edge_index[1]
    pad = EPAD - E
    srcp = jnp.concatenate([src, jnp.zeros((pad,), jnp.int32)])
    dstp = jnp.concatenate([dst, jnp.full((pad,), N, jnp.int32)])
    didx = dstp.reshape(EPAD // ECHUNK, ECHUNK)
    gidx2 = (srcp[None, :] + (jnp.arange(2, dtype=jnp.int32) * N)[:, None]
             ).reshape(2, EPAD // ECHUNK, ECHUNK)
    gidx4 = (srcp[None, :] + (jnp.arange(4, dtype=jnp.int32) * N)[:, None]
             ).reshape(4, EPAD // ECHUNK, ECHUNK)
    bidx3 = batch_index.reshape(N // BR, 1, BR)

    xq = jnp.transpose(x.reshape(N, 2, 128), (1, 0, 2))
    w1v = W1.reshape(2, 128, D)
    w2v = W2.reshape(4, 128, D)
    w3v = W3.reshape(4, 128, D)

    sc_agg2 = _make_sc_agg(2)
    sc_agg4 = _make_sc_agg(4)
    z1 = sc_agg2(xq.reshape(2 * N, CHUNK), gidx2, didx)
    h1 = _layer_mm(z1, w1v, b1.reshape(1, D), 2)
    z2 = sc_agg4(h1.reshape(4 * N, CHUNK), gidx4, didx)
    h2 = _layer_mm(z2, w2v, b2.reshape(1, D), 4)
    z3 = sc_agg4(h2.reshape(4 * N, CHUNK), gidx4, didx)
    sums = _layer3_pool(z3, w3v, b3.reshape(1, D), bidx3, 4)
    return _post(sums, P1, pb1, P2, pb2)


# speedup vs baseline: 1.2013x; 1.2013x over previous
import functools

import jax
import jax.numpy as jnp
from jax import lax
from jax.experimental import pallas as pl
from jax.experimental.pallas import tpu as pltpu
from jax.experimental.pallas import tpu_sc as plsc

N = 10000
E = 160000
DIN = 256
D = 512
G = 64

NC = 2
NS = 16
CHUNK = 128
ECHUNK = 32
EPT = 10240
EPAD = EPT * NS
NCHUNKS = EPT // ECHUNK
NSEG = 8
SEG = NCHUNKS // NSEG
RPT = 624
REM = N - NS * RPT
SLAB_ROWS = N + 16

BR = 1000
RING = 5


def _sc_agg_body(nq, hflat_hbm, gidx_hbm, didx_hbm, z_hbm,
                 gidx_v, didx_v, gbuf, slab, *sems):
    gsem = sems[:RING]
    ssem = sems[RING:]
    c = lax.axis_index("c")
    s = lax.axis_index("s")
    r0 = s * RPT
    passes = nq // NC
    for p in range(passes):
        q = p * NC + c
        hrow0 = pl.multiple_of(q * N + r0, 8)
        pltpu.sync_copy(hflat_hbm.at[pl.ds(hrow0, RPT)],
                        slab.at[pl.ds(r0, RPT)])

        @pl.when(s == NS - 1)
        def _():
            pltpu.sync_copy(
                hflat_hbm.at[pl.ds(pl.multiple_of(q * N + NS * RPT, 8), REM)],
                slab.at[pl.ds(NS * RPT, REM)])
        plsc.subcore_barrier()

        for seg in range(NSEG):
            segbase = s * NCHUNKS + seg * SEG
            pltpu.sync_copy(gidx_hbm.at[q, pl.ds(segbase, SEG)], gidx_v)
            pltpu.sync_copy(didx_hbm.at[pl.ds(segbase, SEG)], didx_v)
            for r in range(RING):
                pltpu.async_copy(hflat_hbm.at[gidx_v.at[r]], gbuf.at[r],
                                 gsem[r])

            def body(m, carry):
                for r in range(RING):
                    j = m * RING + r
                    pltpu.make_async_copy(hflat_hbm.at[pl.ds(0, ECHUNK)],
                                          gbuf.at[r], gsem[r]).wait()
                    pltpu.async_copy(gbuf.at[r], slab.at[didx_v.at[j]],
                                     ssem[r], add=True)
                for r in range(RING):
                    j = m * RING + r
                    pltpu.make_async_copy(gbuf.at[r], slab.at[pl.ds(0, ECHUNK)],
                                          ssem[r]).wait()
                    jn = jnp.minimum(j + RING, SEG - 1)
                    pltpu.async_copy(hflat_hbm.at[gidx_v.at[jn]], gbuf.at[r],
                                     gsem[r])
                return carry

            lax.fori_loop(0, SEG // RING, body, 0)
            for r in range(RING):
                pltpu.make_async_copy(hflat_hbm.at[pl.ds(0, ECHUNK)],
                                      gbuf.at[r], gsem[r]).wait()
        plsc.subcore_barrier()
        pltpu.sync_copy(slab.at[pl.ds(r0, RPT)], z_hbm.at[q, pl.ds(r0, RPT)])

        @pl.when(s == NS - 1)
        def _():
            pltpu.sync_copy(slab.at[pl.ds(NS * RPT, REM)],
                            z_hbm.at[q, pl.ds(NS * RPT, REM)])


@functools.cache
def _make_sc_agg(nq):
    mesh = plsc.VectorSubcoreMesh(core_axis_name="c", subcore_axis_name="s",
                                  num_cores=NC, num_subcores=NS)
    return functools.partial(
        pl.kernel,
        out_type=jax.ShapeDtypeStruct((nq, N, CHUNK), jnp.float32),
        mesh=mesh,
        scratch_types=[
            pltpu.VMEM((SEG, ECHUNK), jnp.int32),
            pltpu.VMEM((SEG, ECHUNK), jnp.int32),
            pltpu.VMEM((RING, ECHUNK, CHUNK), jnp.float32),
            pltpu.VMEM_SHARED((SLAB_ROWS, CHUNK), jnp.float32),
        ] + [pltpu.SemaphoreType.DMA] * (2 * RING) + [
        ],
    )(functools.partial(_sc_agg_body, nq))


def _mm_body(z_ref, w_ref, b_ref, out_ref, *, nq):
    s = jnp.dot(z_ref[0], w_ref[0], preferred_element_type=jnp.float32)
    for q in range(1, nq):
        s += jnp.dot(z_ref[q], w_ref[q], preferred_element_type=jnp.float32)
    h = jnp.maximum(s + b_ref[...], 0.0)
    for qo in range(D // 128):
        out_ref[qo] = h[:, qo * 128:(qo + 1) * 128]


def _layer_mm(z, wv, b2d, nq):
    return pl.pallas_call(
        functools.partial(_mm_body, nq=nq),
        grid=(N // BR,),
        in_specs=[
            pl.BlockSpec((nq, BR, 128), lambda i: (0, i, 0)),
            pl.BlockSpec((nq, 128, D), lambda i: (0, 0, 0)),
            pl.BlockSpec((1, D), lambda i: (0, 0)),
        ],
        out_specs=pl.BlockSpec((D // 128, BR, 128), lambda i: (0, i, 0)),
        out_shape=jax.ShapeDtypeStruct((D // 128, N, 128), jnp.float32),
    )(z, wv, b2d)


def _mm3_body(z_ref, w_ref, b_ref, bi_ref, out_ref, *, nq):
    s = jnp.dot(z_ref[0], w_ref[0], preferred_element_type=jnp.float32)
    for q in range(1, nq):
        s += jnp.dot(z_ref[q], w_ref[q], preferred_element_type=jnp.float32)
    h = jnp.maximum(s + b_ref[...], 0.0)
    hext = jnp.concatenate([h, jnp.ones((BR, 128), jnp.float32)], axis=1)
    bidx = bi_ref[0, 0, :]
    oh = (lax.broadcasted_iota(jnp.int32, (G, BR), 0) == bidx[None, :])
    contrib = jnp.dot(oh.astype(jnp.float32), hext,
                      preferred_element_type=jnp.float32)
    i = pl.program_id(0)

    @pl.when(i == 0)
    def _():
        out_ref[...] = contrib

    @pl.when(i != 0)
    def _():
        out_ref[...] = out_ref[...] + contrib


def _layer3_pool(z, wv, b2d, bidx3, nq):
    return pl.pallas_call(
        functools.partial(_mm3_body, nq=nq),
        grid=(N // BR,),
        in_specs=[
            pl.BlockSpec((nq, BR, 128), lambda i: (0, i, 0)),
            pl.BlockSpec((nq, 128, D), lambda i: (0, 0, 0)),
            pl.BlockSpec((1, D), lambda i: (0, 0)),
            pl.BlockSpec((1, 1, BR), lambda i: (i, 0, 0)),
        ],
        out_specs=pl.BlockSpec((G, D + 128), lambda i: (0, 0)),
        out_shape=jax.ShapeDtypeStruct((G, D + 128), jnp.float32),
    )(z, wv, b2d, bidx3)


def _post_body(s_ref, p1_ref, pb1_ref, p2_ref, pb2_ref, out_ref):
    sums = s_ref[:, :D]
    counts = s_ref[:, D:]
    recip = 1.0 / jnp.maximum(counts, 1.0)
    recip_full = jnp.concatenate([recip] * (D // 128), axis=1)
    pooled = sums * recip_full
    t = jnp.maximum(
        jnp.dot(pooled, p1_ref[...], preferred_element_type=jnp.float32)
        + pb1_ref[...], 0.0)
    out_ref[...] = jnp.maximum(
        jnp.dot(t, p2_ref[...], preferred_element_type=jnp.float32)
        + pb2_ref[...], 0.0)


def _post(sums, p1, pb1, p2, pb2):
    return pl.pallas_call(
        _post_body,
        out_shape=jax.ShapeDtypeStruct((G, D), jnp.float32),
    )(sums, p1, pb1.reshape(1, D), p2, pb2.reshape(1, D))


def kernel(x, edge_index, batch_index, W1, b1, W2, b2, W3, b3, P1, pb1, P2, pb2):
    src = edge_index[0]
    dst = edge_index[1]
    pad = EPAD - E
    srcp = jnp.concatenate([src, jnp.zeros((pad,), jnp.int32)])
    dstp = jnp.concatenate([dst, jnp.full((pad,), N, jnp.int32)])
    didx = dstp.reshape(EPAD // ECHUNK, ECHUNK)
    gidx2 = (srcp[None, :] + (jnp.arange(2, dtype=jnp.int32) * N)[:, None]
             ).reshape(2, EPAD // ECHUNK, ECHUNK)
    gidx4 = (srcp[None, :] + (jnp.arange(4, dtype=jnp.int32) * N)[:, None]
             ).reshape(4, EPAD // ECHUNK, ECHUNK)
    bidx3 = batch_index.reshape(N // BR, 1, BR)

    xq = jnp.transpose(x.reshape(N, 2, 128), (1, 0, 2))
    w1v = W1.reshape(2, 128, D)
    w2v = W2.reshape(4, 128, D)
    w3v = W3.reshape(4, 128, D)

    sc_agg2 = _make_sc_agg(2)
    sc_agg4 = _make_sc_agg(4)
    z1 = sc_agg2(xq.reshape(2 * N, CHUNK), gidx2, didx)
    h1 = _layer_mm(z1, w1v, b1.reshape(1, D), 2)
    z2 = sc_agg4(h1.reshape(4 * N, CHUNK), gidx4, didx)
    h2 = _layer_mm(z2, w2v, b2.reshape(1, D), 4)
    z3 = sc_agg4(h2.reshape(4 * N, CHUNK), gidx4, didx)
    sums = _layer3_pool(z3, w3v, b3.reshape(1, D), bidx3, 4)
    return _post(sums, P1, pb1, P2, pb2)


# speedup vs baseline: 1.2481x; 1.0390x over previous
import functools

import jax
import jax.numpy as jnp
from jax import lax
from jax.experimental import pallas as pl
from jax.experimental.pallas import tpu as pltpu
from jax.experimental.pallas import tpu_sc as plsc

N = 10000
E = 160000
DIN = 256
D = 512
G = 64

NC = 2
NS = 16
CHUNK = 128
ECHUNK = 64
EPT = 10240
EPAD = EPT * NS
NCHUNKS = EPT // ECHUNK
NSEG = 4
SEG = NCHUNKS // NSEG
RPT = 624
REM = N - NS * RPT
SLAB_ROWS = N + 16

BR = 1000
RING = 4


def _sc_agg_body(nq, hflat_hbm, gidx_hbm, didx_hbm, z_hbm,
                 gidx_v, didx_v, gbuf, slab, *sems):
    gsem = sems[:RING]
    ssem = sems[RING:]
    c = lax.axis_index("c")
    s = lax.axis_index("s")
    r0 = s * RPT
    passes = nq // NC
    for p in range(passes):
        q = p * NC + c
        hrow0 = pl.multiple_of(q * N + r0, 8)
        pltpu.sync_copy(hflat_hbm.at[pl.ds(hrow0, RPT)],
                        slab.at[pl.ds(r0, RPT)])

        @pl.when(s == NS - 1)
        def _():
            pltpu.sync_copy(
                hflat_hbm.at[pl.ds(pl.multiple_of(q * N + NS * RPT, 8), REM)],
                slab.at[pl.ds(NS * RPT, REM)])
        plsc.subcore_barrier()

        for seg in range(NSEG):
            segbase = s * NCHUNKS + seg * SEG
            pltpu.sync_copy(gidx_hbm.at[q, pl.ds(segbase, SEG)], gidx_v)
            pltpu.sync_copy(didx_hbm.at[pl.ds(segbase, SEG)], didx_v)
            for r in range(RING):
                pltpu.async_copy(hflat_hbm.at[gidx_v.at[r]], gbuf.at[r],
                                 gsem[r])

            def body(m, carry):
                for r in range(RING):
                    j = m * RING + r
                    pltpu.make_async_copy(hflat_hbm.at[pl.ds(0, ECHUNK)],
                                          gbuf.at[r], gsem[r]).wait()
                    pltpu.async_copy(gbuf.at[r], slab.at[didx_v.at[j]],
                                     ssem[r], add=True)
                for r in range(RING):
                    j = m * RING + r
                    pltpu.make_async_copy(gbuf.at[r], slab.at[pl.ds(0, ECHUNK)],
                                          ssem[r]).wait()
                    jn = jnp.minimum(j + RING, SEG - 1)
                    pltpu.async_copy(hflat_hbm.at[gidx_v.at[jn]], gbuf.at[r],
                                     gsem[r])
                return carry

            lax.fori_loop(0, SEG // RING, body, 0)
            for r in range(RING):
                pltpu.make_async_copy(hflat_hbm.at[pl.ds(0, ECHUNK)],
                                      gbuf.at[r], gsem[r]).wait()
        plsc.subcore_barrier()
        pltpu.sync_copy(slab.at[pl.ds(r0, RPT)], z_hbm.at[q, pl.ds(r0, RPT)])

        @pl.when(s == NS - 1)
        def _():
            pltpu.sync_copy(slab.at[pl.ds(NS * RPT, REM)],
                            z_hbm.at[q, pl.ds(NS * RPT, REM)])


@functools.cache
def _make_sc_agg(nq):
    mesh = plsc.VectorSubcoreMesh(core_axis_name="c", subcore_axis_name="s",
                                  num_cores=NC, num_subcores=NS)
    return functools.partial(
        pl.kernel,
        out_type=jax.ShapeDtypeStruct((nq, N, CHUNK), jnp.float32),
        mesh=mesh,
        scratch_types=[
            pltpu.VMEM((SEG, ECHUNK), jnp.int32),
            pltpu.VMEM((SEG, ECHUNK), jnp.int32),
            pltpu.VMEM((RING, ECHUNK, CHUNK), jnp.float32),
            pltpu.VMEM_SHARED((SLAB_ROWS, CHUNK), jnp.float32),
        ] + [pltpu.SemaphoreType.DMA] * (2 * RING) + [
        ],
    )(functools.partial(_sc_agg_body, nq))


def _mm_body(z_ref, w_ref, b_ref, out_ref, *, nq):
    s = jnp.dot(z_ref[0], w_ref[0], preferred_element_type=jnp.float32)
    for q in range(1, nq):
        s += jnp.dot(z_ref[q], w_ref[q], preferred_element_type=jnp.float32)
    h = jnp.maximum(s + b_ref[...], 0.0)
    for qo in range(D // 128):
        out_ref[qo] = h[:, qo * 128:(qo + 1) * 128]


def _layer_mm(z, wv, b2d, nq):
    return pl.pallas_call(
        functools.partial(_mm_body, nq=nq),
        grid=(N // BR,),
        in_specs=[
            pl.BlockSpec((nq, BR, 128), lambda i: (0, i, 0)),
            pl.BlockSpec((nq, 128, D), lambda i: (0, 0, 0)),
            pl.BlockSpec((1, D), lambda i: (0, 0)),
        ],
        out_specs=pl.BlockSpec((D // 128, BR, 128), lambda i: (0, i, 0)),
        out_shape=jax.ShapeDtypeStruct((D // 128, N, 128), jnp.float32),
    )(z, wv, b2d)


def _mm3_body(z_ref, w_ref, b_ref, bi_ref, out_ref, *, nq):
    s = jnp.dot(z_ref[0], w_ref[0], preferred_element_type=jnp.float32)
    for q in range(1, nq):
        s += jnp.dot(z_ref[q], w_ref[q], preferred_element_type=jnp.float32)
    h = jnp.maximum(s + b_ref[...], 0.0)
    hext = jnp.concatenate([h, jnp.ones((BR, 128), jnp.float32)], axis=1)
    bidx = bi_ref[0, 0, :]
    oh = (lax.broadcasted_iota(jnp.int32, (G, BR), 0) == bidx[None, :])
    contrib = jnp.dot(oh.astype(jnp.float32), hext,
                      preferred_element_type=jnp.float32)
    i = pl.program_id(0)

    @pl.when(i == 0)
    def _():
        out_ref[...] = contrib

    @pl.when(i != 0)
    def _():
        out_ref[...] = out_ref[...] + contrib


def _layer3_pool(z, wv, b2d, bidx3, nq):
    return pl.pallas_call(
        functools.partial(_mm3_body, nq=nq),
        grid=(N // BR,),
        in_specs=[
            pl.BlockSpec((nq, BR, 128), lambda i: (0, i, 0)),
            pl.BlockSpec((nq, 128, D), lambda i: (0, 0, 0)),
            pl.BlockSpec((1, D), lambda i: (0, 0)),
            pl.BlockSpec((1, 1, BR), lambda i: (i, 0, 0)),
        ],
        out_specs=pl.BlockSpec((G, D + 128), lambda i: (0, 0)),
        out_shape=jax.ShapeDtypeStruct((G, D + 128), jnp.float32),
    )(z, wv, b2d, bidx3)


def _post_body(s_ref, p1_ref, pb1_ref, p2_ref, pb2_ref, out_ref):
    sums = s_ref[:, :D]
    counts = s_ref[:, D:]
    recip = 1.0 / jnp.maximum(counts, 1.0)
    recip_full = jnp.concatenate([recip] * (D // 128), axis=1)
    pooled = sums * recip_full
    t = jnp.maximum(
        jnp.dot(pooled, p1_ref[...], preferred_element_type=jnp.float32)
        + pb1_ref[...], 0.0)
    out_ref[...] = jnp.maximum(
        jnp.dot(t, p2_ref[...], preferred_element_type=jnp.float32)
        + pb2_ref[...], 0.0)


def _post(sums, p1, pb1, p2, pb2):
    return pl.pallas_call(
        _post_body,
        out_shape=jax.ShapeDtypeStruct((G, D), jnp.float32),
    )(sums, p1, pb1.reshape(1, D), p2, pb2.reshape(1, D))


def kernel(x, edge_index, batch_index, W1, b1, W2, b2, W3, b3, P1, pb1, P2, pb2):
    src = edge_index[0]
    dst = edge_index[1]
    pad = EPAD - E
    srcp = jnp.concatenate([src, jnp.zeros((pad,), jnp.int32)])
    dstp = jnp.concatenate([dst, jnp.full((pad,), N, jnp.int32)])
    didx = dstp.reshape(EPAD // ECHUNK, ECHUNK)
    gidx2 = (srcp[None, :] + (jnp.arange(2, dtype=jnp.int32) * N)[:, None]
             ).reshape(2, EPAD // ECHUNK, ECHUNK)
    gidx4 = (srcp[None, :] + (jnp.arange(4, dtype=jnp.int32) * N)[:, None]
             ).reshape(4, EPAD // ECHUNK, ECHUNK)
    bidx3 = batch_index.reshape(N // BR, 1, BR)

    xq = jnp.transpose(x.reshape(N, 2, 128), (1, 0, 2))
    w1v = W1.reshape(2, 128, D)
    w2v = W2.reshape(4, 128, D)
    w3v = W3.reshape(4, 128, D)

    sc_agg2 = _make_sc_agg(2)
    sc_agg4 = _make_sc_agg(4)
    z1 = sc_agg2(xq.reshape(2 * N, CHUNK), gidx2, didx)
    h1 = _layer_mm(z1, w1v, b1.reshape(1, D), 2)
    z2 = sc_agg4(h1.reshape(4 * N, CHUNK), gidx4, didx)
    h2 = _layer_mm(z2, w2v, b2.reshape(1, D), 4)
    z3 = sc_agg4(h2.reshape(4 * N, CHUNK), gidx4, didx)
    sums = _layer3_pool(z3, w3v, b3.reshape(1, D), bidx3, 4)
    return _post(sums, P1, pb1, P2, pb2)
